# Initial kernel scaffold; baseline (speedup 1.0000x reference)
import jax
import jax.numpy as jnp
from jax.experimental import pallas as pl


def kernel(x, pos_table):
    raise NotImplementedError("write your pallas kernel here")


# speedup vs baseline: 1.6818x; 1.6818x over previous
import jax
import jax.numpy as jnp
from jax.experimental import pallas as pl
from jax.experimental.pallas import tpu as pltpu

_TB = 1024


def _add_kernel(x_ref, tab_ref, o_ref):
    o_ref[...] = x_ref[...] + tab_ref[...]


def kernel(x, pos_table):
    B, T, D = x.shape
    grid = (T // _TB, B)
    return pl.pallas_call(
        _add_kernel,
        grid=grid,
        in_specs=[
            pl.BlockSpec((1, _TB, D), lambda t, b: (b, t, 0)),
            pl.BlockSpec((_TB, D), lambda t, b: (t, 0)),
        ],
        out_specs=pl.BlockSpec((1, _TB, D), lambda t, b: (b, t, 0)),
        out_shape=jax.ShapeDtypeStruct((B, T, D), x.dtype),
    )(x, pos_table[:T])


# speedup vs baseline: 1.7954x; 1.0675x over previous
import jax
import jax.numpy as jnp
from jax.experimental import pallas as pl
from jax.experimental.pallas import tpu as pltpu

_TB = 2048


def _add_kernel(x_ref, tab_ref, o_ref):
    o_ref[...] = x_ref[...] + tab_ref[...]


def kernel(x, pos_table):
    B, T, D = x.shape
    grid = (T // _TB, B)
    return pl.pallas_call(
        _add_kernel,
        grid=grid,
        in_specs=[
            pl.BlockSpec((1, _TB, D), lambda t, b: (b, t, 0)),
            pl.BlockSpec((_TB, D), lambda t, b: (t, 0)),
        ],
        out_specs=pl.BlockSpec((1, _TB, D), lambda t, b: (b, t, 0)),
        out_shape=jax.ShapeDtypeStruct((B, T, D), x.dtype),
    )(x, pos_table[:T])
